# baseline (speedup 1.0000x reference)
import functools

import jax
import jax.numpy as jnp
import numpy as np
from jax import lax
from jax.experimental import pallas as pl
from jax.experimental.pallas import tpu as pltpu
from jax.sharding import Mesh, PartitionSpec as P

try:
    from jax.experimental.shard_map import shard_map as _shard_map
except ImportError:
    _shard_map = jax.shard_map


def _round_up(x, m):
    return (x + m - 1) // m * m


def _sig(x):
    e = jnp.exp(-jnp.abs(x))
    inv = 1.0 / (1.0 + e)
    return jnp.where(x >= 0, inv, e * inv)


def _softplus(x):
    return jnp.maximum(x, 0.0) + jnp.log(1.0 + jnp.exp(-jnp.abs(x)))


def _ntm_kernel(
    x_ref,
    reads0_ref,
    h0_ref,
    c0_ref,
    mem0_ref,
    w0_ref,
    lstm_wT_ref,
    lstm_bT_ref,
    head_wT_ref,
    head_bT_ref,
    out_wT_ref,
    out_bT_ref,
    y_ref,
    h_ref, c_ref, mem_ref, w_ref, reads_ref,
    *,
    num_heads, ctrl, mem_m, mem_n, in_size, total_t, tt_chunk,
):
    C = ctrl
    M = mem_m
    nH = num_heads
    IN = in_size
    f32 = jnp.float32
    KB = 2 * nH * M
    SB = KB
    EB = SB + 128
    AB = EB + nH * M
    tc = pl.program_id(1)

    @pl.when(tc == 0)
    def _init():
        h_ref[...] = h0_ref[...]
        c_ref[...] = c0_ref[...]
        mem_ref[...] = mem0_ref[...]
        w_ref[...] = w0_ref[...]
        reads_ref[...] = reads0_ref[...]

    w_x = lstm_wT_ref[:, 0:IN]
    w_r = lstm_wT_ref[:, IN:IN + nH * M]
    w_h = lstm_wT_ref[:, IN + nH * M:IN + nH * M + C]
    lstm_b = lstm_bT_ref[...]
    head_w = head_wT_ref[...]
    head_b = head_bT_ref[...]
    ow_h = out_wT_ref[:, 0:C]
    ow_r = out_wT_ref[:, C:C + nH * M]
    out_b = out_bT_ref[...]

    def step(tt, carry):
        h, c, reads = carry
        x = x_ref[tt]

        gates = (jnp.dot(w_x, x, preferred_element_type=f32)
                 + jnp.dot(w_r, reads, preferred_element_type=f32)
                 + jnp.dot(w_h, h, preferred_element_type=f32)
                 + lstm_b)
        i_g = _sig(gates[0:C])
        f_g = _sig(gates[C:2 * C])
        g_g = jnp.tanh(gates[2 * C:3 * C])
        o_g = _sig(gates[3 * C:4 * C])
        c_new = f_g * c + i_g * g_g
        h_new = o_g * jnp.tanh(c_new)

        z = jnp.dot(head_w, c_new, preferred_element_type=f32) + head_b

        beta_a = _softplus(z[SB:SB + 8])
        gate_a = _sig(z[SB + 8:SB + 16])
        s0 = z[SB + 16:SB + 24]
        s1 = z[SB + 24:SB + 32]
        s2 = z[SB + 32:SB + 40]
        m3 = jnp.maximum(jnp.maximum(s0, s1), s2)
        e0 = jnp.exp(s0 - m3)
        e1 = jnp.exp(s1 - m3)
        e2 = jnp.exp(s2 - m3)
        den = e0 + e1 + e2
        s0, s1, s2 = e0 / den, e1 / den, e2 / den
        gamma_a = 1.0 + _softplus(z[SB + 40:SB + 48])
        erase_a = _sig(z[EB:EB + nH * M])
        add_a = jnp.tanh(z[AB:AB + nH * M])

        mem = mem_ref[...]
        mem_nrm = jnp.sqrt(jnp.sum(mem * mem, axis=0))

        new_reads = []
        for p in range(nH):
            w_wr = None
            mem = mem_ref[...]
            for j in range(2):
                s = 2 * p + j
                key = z[s * M:(s + 1) * M]
                kn = jnp.sqrt(jnp.sum(key * key, axis=0, keepdims=True))
                dots = jnp.sum(key[:, None, :] * mem, axis=0)
                sim = dots / jnp.maximum(kn * mem_nrm, 1e-8)
                a = beta_a[s:s + 1] * sim
                a = a - jnp.max(a, axis=0, keepdims=True)
                e = jnp.exp(a)
                cw = e / jnp.sum(e, axis=0, keepdims=True)
                g = gate_a[s:s + 1]
                iw = g * cw + (1.0 - g) * w_ref[s]
                iw_m1 = jnp.concatenate([iw[-1:], iw[:-1]], axis=0)
                iw_p1 = jnp.concatenate([iw[1:], iw[:1]], axis=0)
                shifted = (s0[s:s + 1] * iw_m1
                           + s1[s:s + 1] * iw
                           + s2[s:s + 1] * iw_p1)
                sharp = jnp.exp(gamma_a[s:s + 1]
                                * jnp.log(jnp.maximum(shifted, 1e-30)))
                wv = sharp / (jnp.sum(sharp, axis=0, keepdims=True) + 1e-16)
                w_ref[s] = wv
                if j == 0:
                    new_reads.append(jnp.sum(wv[None, :, :] * mem, axis=1))
                else:
                    w_wr = wv

            er = erase_a[p * M:(p + 1) * M]
            ad = add_a[p * M:(p + 1) * M]
            ww = w_wr[None, :, :]
            mem_ref[...] = mem - ww * (mem * er[:, None, :] - ad[:, None, :])
            if p + 1 < nH:
                mem = mem_ref[...]
                mem_nrm = jnp.sqrt(jnp.sum(mem * mem, axis=0))

        reads_new = jnp.concatenate(new_reads, axis=0)
        y = _sig(jnp.dot(ow_h, h_new, preferred_element_type=f32)
                 + jnp.dot(ow_r, reads_new, preferred_element_type=f32)
                 + out_b)
        y_ref[tt] = y
        return (h_new, c_new, reads_new)

    if total_t % tt_chunk == 0:
        n_steps = tt_chunk
    else:
        n_steps = jnp.minimum(tt_chunk, total_t - tc * tt_chunk)

    carry = (h_ref[...], c_ref[...], reads_ref[...])
    h, c, reads = lax.fori_loop(0, n_steps, step, carry)

    h_ref[...] = h
    c_ref[...] = c
    reads_ref[...] = reads


def _repack_head(head_w, head_b, num_heads, mem_m):
    M = mem_m
    HSZ = 3 * M + 6
    nS = 2 * num_heads

    def repack(a):
        keys = jnp.concatenate(
            [a[:, s * HSZ:s * HSZ + M] for s in range(nS)], axis=1)
        scal = jnp.concatenate(
            [jnp.concatenate([a[:, s * HSZ + M + k:s * HSZ + M + k + 1]
                              for s in range(nS)], axis=1)
             for k in range(6)], axis=1)
        scal = jnp.pad(scal, ((0, 0), (0, 128 - 6 * nS)))
        erase = jnp.concatenate(
            [a[:, (2 * p + 1) * HSZ + M + 6:(2 * p + 1) * HSZ + 2 * M + 6]
             for p in range(num_heads)], axis=1)
        add = jnp.concatenate(
            [a[:, (2 * p + 1) * HSZ + 2 * M + 6:(2 * p + 1) * HSZ + 3 * M + 6]
             for p in range(num_heads)], axis=1)
        return jnp.concatenate([keys, scal, erase, add], axis=1)

    return repack(head_w), repack(head_b)


def kernel(lstm_w, lstm_b, head_w, head_b, out_w, out_b,
           h, c, memory, prev_w, prev_reads, x_seq):
    f32 = jnp.float32
    T, B, IN = x_seq.shape
    C = h.shape[1]
    _, N, M = memory.shape
    nH = prev_reads.shape[0]
    OP = out_b.shape[1]
    output_size = 128

    BT = 128
    ndev = 2 if len(jax.devices()) >= 2 and B >= 256 else 1
    Bp = _round_up(B, BT * ndev)
    TT = min(16, T)
    n_chunks = pl.cdiv(T, TT)
    Tp = n_chunks * TT

    hw, hb = _repack_head(head_w.astype(f32), head_b.astype(f32), nH, M)
    HPn = hw.shape[1]
    lstm_in = IN + nH * M + C

    lstm_wT = lstm_w.astype(f32).T
    head_wT = hw.T
    out_wT = out_w.astype(f32).T
    lstm_bT = jnp.broadcast_to(lstm_b.T, (4 * C, BT))
    head_bT = jnp.broadcast_to(hb.T, (HPn, BT))
    out_bT = jnp.broadcast_to(out_b.T, (OP, BT))

    def pad_b(a, axis):
        if a.shape[axis] == Bp:
            return a.astype(f32)
        widths = [(0, 0)] * a.ndim
        widths[axis] = (0, Bp - a.shape[axis])
        return jnp.pad(a.astype(f32), widths)

    x_p = pad_b(jnp.pad(x_seq.astype(f32), ((0, Tp - T), (0, 0), (0, 0)))
                .transpose(0, 2, 1), 2)
    reads_p = pad_b(jnp.transpose(prev_reads, (0, 2, 1))
                    .reshape(nH * M, B), 1)
    h_p = pad_b(h.T, 1)
    c_p = pad_b(c.T, 1)
    mem_p = pad_b(jnp.transpose(memory, (2, 1, 0)), 2)
    w_p = pad_b(jnp.transpose(prev_w, (0, 2, 1)), 2)

    def run_local(x_l, reads_l, h_l, c_l, mem_l, w_l,
                  lw_l, lb_l, hww_l, hbb_l, oww_l, obb_l):
        Bl = x_l.shape[2]
        nb = Bl // BT
        grid = (nb, n_chunks)
        in_specs = [
            pl.BlockSpec((TT, IN, BT), lambda b, t: (t, 0, b)),
            pl.BlockSpec((nH * M, BT), lambda b, t: (0, b)),
            pl.BlockSpec((C, BT), lambda b, t: (0, b)),
            pl.BlockSpec((C, BT), lambda b, t: (0, b)),
            pl.BlockSpec((M, N, BT), lambda b, t: (0, 0, b)),
            pl.BlockSpec((2 * nH, N, BT), lambda b, t: (0, 0, b)),
            pl.BlockSpec((4 * C, lstm_in), lambda b, t: (0, 0)),
            pl.BlockSpec((4 * C, BT), lambda b, t: (0, 0)),
            pl.BlockSpec((HPn, C), lambda b, t: (0, 0)),
            pl.BlockSpec((HPn, BT), lambda b, t: (0, 0)),
            pl.BlockSpec((OP, C + nH * M), lambda b, t: (0, 0)),
            pl.BlockSpec((OP, BT), lambda b, t: (0, 0)),
        ]
        out_specs = (
            pl.BlockSpec((TT, OP, BT), lambda b, t: (t, 0, b)),
            pl.BlockSpec((C, BT), lambda b, t: (0, b)),
            pl.BlockSpec((C, BT), lambda b, t: (0, b)),
            pl.BlockSpec((M, N, BT), lambda b, t: (0, 0, b)),
            pl.BlockSpec((2 * nH, N, BT), lambda b, t: (0, 0, b)),
            pl.BlockSpec((nH * M, BT), lambda b, t: (0, b)),
        )
        out_shapes = (
            jax.ShapeDtypeStruct((Tp, OP, Bl), f32),
            jax.ShapeDtypeStruct((C, Bl), f32),
            jax.ShapeDtypeStruct((C, Bl), f32),
            jax.ShapeDtypeStruct((M, N, Bl), f32),
            jax.ShapeDtypeStruct((2 * nH, N, Bl), f32),
            jax.ShapeDtypeStruct((nH * M, Bl), f32),
        )
        fn = pl.pallas_call(
            functools.partial(
                _ntm_kernel,
                num_heads=nH, ctrl=C, mem_m=M, mem_n=N, in_size=IN,
                total_t=T, tt_chunk=TT),
            grid=grid,
            in_specs=in_specs,
            out_specs=out_specs,
            out_shape=out_shapes,
            compiler_params=pltpu.CompilerParams(
                dimension_semantics=("parallel", "arbitrary"),
                vmem_limit_bytes=60 * 1024 * 1024),
        )
        return fn(x_l, reads_l, h_l, c_l, mem_l, w_l,
                  lw_l, lb_l, hww_l, hbb_l, oww_l, obb_l)

    if ndev > 1:
        mesh = Mesh(np.array(jax.devices()[:ndev]), ("d",))
        sh_b2 = P(None, "d")
        sh_b3 = P(None, None, "d")
        rep = P()
        run = _shard_map(
            run_local, mesh=mesh,
            in_specs=(sh_b3, sh_b2, sh_b2, sh_b2, sh_b3, sh_b3,
                      rep, rep, rep, rep, rep, rep),
            out_specs=(sh_b3, sh_b2, sh_b2, sh_b3, sh_b3, sh_b2),
            check_rep=False)
    else:
        run = run_local

    y_seq, h_new, c_new, mem_new, w_new, reads_new = run(
        x_p, reads_p, h_p, c_p, mem_p, w_p,
        lstm_wT, lstm_bT, head_wT, head_bT, out_wT, out_bT)

    new_state = {
        "h": h_new[:, :B].T,
        "c": c_new[:, :B].T,
        "memory": jnp.transpose(mem_new[:, :, :B], (2, 1, 0)),
        "prev_w": jnp.transpose(w_new[:, :, :B], (0, 2, 1)),
        "prev_reads": jnp.transpose(
            reads_new[:, :B].reshape(nH, M, B), (0, 2, 1)),
    }
    return (jnp.transpose(y_seq[:T, :output_size, :B], (0, 2, 1)), new_state)


# speedup vs baseline: 1.0515x; 1.0515x over previous
import functools

import jax
import jax.numpy as jnp
import numpy as np
from jax import lax
from jax.experimental import pallas as pl
from jax.experimental.pallas import tpu as pltpu
from jax.sharding import Mesh, PartitionSpec as P

try:
    from jax.experimental.shard_map import shard_map as _shard_map
except ImportError:
    _shard_map = jax.shard_map


def _round_up(x, m):
    return (x + m - 1) // m * m


def _sig(x):
    e = jnp.exp(-jnp.abs(x))
    inv = 1.0 / (1.0 + e)
    return jnp.where(x >= 0, inv, e * inv)


def _softplus(x):
    return jnp.maximum(x, 0.0) + jnp.log(1.0 + jnp.exp(-jnp.abs(x)))


def _ntm_kernel(
    x_ref,
    reads0_ref,
    h0_ref,
    c0_ref,
    mem0_ref,
    w0_ref,
    lstm_wT_ref,
    lstm_bT_ref,
    head_wT_ref,
    head_bT_ref,
    out_wT_ref,
    out_bT_ref,
    y_ref,
    h_ref, c_ref, mem_ref, w_ref, reads_ref,
    *,
    num_heads, ctrl, mem_m, mem_n, in_size, total_t, tt_chunk,
):
    C = ctrl
    M = mem_m
    nH = num_heads
    IN = in_size
    f32 = jnp.float32
    KB = 2 * nH * M
    SB = KB
    EB = SB + 128
    AB = EB + nH * M
    tc = pl.program_id(1)

    @pl.when(tc == 0)
    def _init():
        h_ref[...] = h0_ref[...]
        c_ref[...] = c0_ref[...]
        mem_ref[...] = mem0_ref[...]
        w_ref[...] = w0_ref[...]
        reads_ref[...] = reads0_ref[...]

    w_x = lstm_wT_ref[:, 0:IN]
    w_r = lstm_wT_ref[:, IN:IN + nH * M]
    w_h = lstm_wT_ref[:, IN + nH * M:IN + nH * M + C]
    lstm_b = lstm_bT_ref[...]
    head_w = head_wT_ref[...]
    head_b = head_bT_ref[...]
    ow_h = out_wT_ref[:, 0:C]
    ow_r = out_wT_ref[:, C:C + nH * M]
    out_b = out_bT_ref[...]

    def step(tt, carry):
        h, c, reads = carry
        x = x_ref[tt]

        gates = (jnp.dot(w_x, x, preferred_element_type=f32)
                 + jnp.dot(w_r, reads, preferred_element_type=f32)
                 + jnp.dot(w_h, h, preferred_element_type=f32)
                 + lstm_b)
        i_g = _sig(gates[0:C])
        f_g = _sig(gates[C:2 * C])
        g_g = jnp.tanh(gates[2 * C:3 * C])
        o_g = _sig(gates[3 * C:4 * C])
        c_new = f_g * c + i_g * g_g
        h_new = o_g * jnp.tanh(c_new)

        z = jnp.dot(head_w, c_new, preferred_element_type=f32) + head_b

        beta_a = _softplus(z[SB:SB + 8])
        gate_a = _sig(z[SB + 8:SB + 16])
        s0 = z[SB + 16:SB + 24]
        s1 = z[SB + 24:SB + 32]
        s2 = z[SB + 32:SB + 40]
        m3 = jnp.maximum(jnp.maximum(s0, s1), s2)
        e0 = jnp.exp(s0 - m3)
        e1 = jnp.exp(s1 - m3)
        e2 = jnp.exp(s2 - m3)
        den = e0 + e1 + e2
        s0, s1, s2 = e0 / den, e1 / den, e2 / den
        gamma_a = 1.0 + _softplus(z[SB + 40:SB + 48])
        erase_a = _sig(z[EB:EB + nH * M])
        add_a = jnp.tanh(z[AB:AB + nH * M])

        new_reads = []
        for p in range(nH):
            kr = z[2 * p * M:(2 * p + 1) * M]
            kw = z[(2 * p + 1) * M:(2 * p + 2) * M]

            drs, dws, n2s = [], [], []
            for nc in range(0, mem_n, 8):
                mslc = mem_ref[:, nc:nc + 8, :]
                drs.append(jnp.sum(kr[:, None, :] * mslc, axis=0))
                dws.append(jnp.sum(kw[:, None, :] * mslc, axis=0))
                n2s.append(jnp.sum(mslc * mslc, axis=0))
            dots2 = (jnp.concatenate(drs, axis=0),
                     jnp.concatenate(dws, axis=0))
            mem_nrm = jnp.sqrt(jnp.concatenate(n2s, axis=0))

            w_pair = []
            for j in range(2):
                s = 2 * p + j
                key = kr if j == 0 else kw
                kn = jnp.sqrt(jnp.sum(key * key, axis=0, keepdims=True))
                sim = dots2[j] / jnp.maximum(kn * mem_nrm, 1e-8)
                a = beta_a[s:s + 1] * sim
                a = a - jnp.max(a, axis=0, keepdims=True)
                e = jnp.exp(a)
                cw = e / jnp.sum(e, axis=0, keepdims=True)
                g = gate_a[s:s + 1]
                iw = g * cw + (1.0 - g) * w_ref[s]
                iw_m1 = jnp.concatenate([iw[-1:], iw[:-1]], axis=0)
                iw_p1 = jnp.concatenate([iw[1:], iw[:1]], axis=0)
                shifted = (s0[s:s + 1] * iw_m1
                           + s1[s:s + 1] * iw
                           + s2[s:s + 1] * iw_p1)
                sharp = jnp.exp(gamma_a[s:s + 1]
                                * jnp.log(jnp.maximum(shifted, 1e-30)))
                wv = sharp / (jnp.sum(sharp, axis=0, keepdims=True) + 1e-16)
                w_ref[s] = wv
                w_pair.append(wv)

            wv_r = w_pair[0]
            rds = [jnp.sum(wv_r[None, :, :] * mem_ref[mc:mc + 8], axis=1)
                   for mc in range(0, M, 8)]
            new_reads.append(jnp.concatenate(rds, axis=0))

            er = erase_a[p * M:(p + 1) * M]
            ad = add_a[p * M:(p + 1) * M]
            ww = w_pair[1][None, :, :]
            for mc in range(0, M, 8):
                mslc = mem_ref[mc:mc + 8]
                mem_ref[mc:mc + 8] = mslc - ww * (
                    mslc * er[mc:mc + 8, None, :] - ad[mc:mc + 8, None, :])

        reads_new = jnp.concatenate(new_reads, axis=0)
        y = _sig(jnp.dot(ow_h, h_new, preferred_element_type=f32)
                 + jnp.dot(ow_r, reads_new, preferred_element_type=f32)
                 + out_b)
        y_ref[tt] = y
        return (h_new, c_new, reads_new)

    if total_t % tt_chunk == 0:
        n_steps = tt_chunk
    else:
        n_steps = jnp.minimum(tt_chunk, total_t - tc * tt_chunk)

    carry = (h_ref[...], c_ref[...], reads_ref[...])
    h, c, reads = lax.fori_loop(0, n_steps, step, carry)

    h_ref[...] = h
    c_ref[...] = c
    reads_ref[...] = reads


def _repack_head(head_w, head_b, num_heads, mem_m):
    M = mem_m
    HSZ = 3 * M + 6
    nS = 2 * num_heads

    def repack(a):
        keys = jnp.concatenate(
            [a[:, s * HSZ:s * HSZ + M] for s in range(nS)], axis=1)
        scal = jnp.concatenate(
            [jnp.concatenate([a[:, s * HSZ + M + k:s * HSZ + M + k + 1]
                              for s in range(nS)], axis=1)
             for k in range(6)], axis=1)
        scal = jnp.pad(scal, ((0, 0), (0, 128 - 6 * nS)))
        erase = jnp.concatenate(
            [a[:, (2 * p + 1) * HSZ + M + 6:(2 * p + 1) * HSZ + 2 * M + 6]
             for p in range(num_heads)], axis=1)
        add = jnp.concatenate(
            [a[:, (2 * p + 1) * HSZ + 2 * M + 6:(2 * p + 1) * HSZ + 3 * M + 6]
             for p in range(num_heads)], axis=1)
        return jnp.concatenate([keys, scal, erase, add], axis=1)

    return repack(head_w), repack(head_b)


def kernel(lstm_w, lstm_b, head_w, head_b, out_w, out_b,
           h, c, memory, prev_w, prev_reads, x_seq):
    f32 = jnp.float32
    T, B, IN = x_seq.shape
    C = h.shape[1]
    _, N, M = memory.shape
    nH = prev_reads.shape[0]
    OP = out_b.shape[1]
    output_size = 128

    BT = 128
    ndev = 2 if len(jax.devices()) >= 2 and B >= 256 else 1
    Bp = _round_up(B, BT * ndev)
    TT = min(16, T)
    n_chunks = pl.cdiv(T, TT)
    Tp = n_chunks * TT

    hw, hb = _repack_head(head_w.astype(f32), head_b.astype(f32), nH, M)
    HPn = hw.shape[1]
    lstm_in = IN + nH * M + C

    lstm_wT = lstm_w.astype(f32).T
    head_wT = hw.T
    out_wT = out_w.astype(f32).T
    lstm_bT = jnp.broadcast_to(lstm_b.T, (4 * C, BT))
    head_bT = jnp.broadcast_to(hb.T, (HPn, BT))
    out_bT = jnp.broadcast_to(out_b.T, (OP, BT))

    def pad_b(a, axis):
        if a.shape[axis] == Bp:
            return a.astype(f32)
        widths = [(0, 0)] * a.ndim
        widths[axis] = (0, Bp - a.shape[axis])
        return jnp.pad(a.astype(f32), widths)

    x_p = pad_b(jnp.pad(x_seq.astype(f32), ((0, Tp - T), (0, 0), (0, 0)))
                .transpose(0, 2, 1), 2)
    reads_p = pad_b(jnp.transpose(prev_reads, (0, 2, 1))
                    .reshape(nH * M, B), 1)
    h_p = pad_b(h.T, 1)
    c_p = pad_b(c.T, 1)
    mem_p = pad_b(jnp.transpose(memory, (2, 1, 0)), 2)
    w_p = pad_b(jnp.transpose(prev_w, (0, 2, 1)), 2)

    def run_local(x_l, reads_l, h_l, c_l, mem_l, w_l,
                  lw_l, lb_l, hww_l, hbb_l, oww_l, obb_l):
        Bl = x_l.shape[2]
        nb = Bl // BT
        grid = (nb, n_chunks)
        in_specs = [
            pl.BlockSpec((TT, IN, BT), lambda b, t: (t, 0, b)),
            pl.BlockSpec((nH * M, BT), lambda b, t: (0, b)),
            pl.BlockSpec((C, BT), lambda b, t: (0, b)),
            pl.BlockSpec((C, BT), lambda b, t: (0, b)),
            pl.BlockSpec((M, N, BT), lambda b, t: (0, 0, b)),
            pl.BlockSpec((2 * nH, N, BT), lambda b, t: (0, 0, b)),
            pl.BlockSpec((4 * C, lstm_in), lambda b, t: (0, 0)),
            pl.BlockSpec((4 * C, BT), lambda b, t: (0, 0)),
            pl.BlockSpec((HPn, C), lambda b, t: (0, 0)),
            pl.BlockSpec((HPn, BT), lambda b, t: (0, 0)),
            pl.BlockSpec((OP, C + nH * M), lambda b, t: (0, 0)),
            pl.BlockSpec((OP, BT), lambda b, t: (0, 0)),
        ]
        out_specs = (
            pl.BlockSpec((TT, OP, BT), lambda b, t: (t, 0, b)),
            pl.BlockSpec((C, BT), lambda b, t: (0, b)),
            pl.BlockSpec((C, BT), lambda b, t: (0, b)),
            pl.BlockSpec((M, N, BT), lambda b, t: (0, 0, b)),
            pl.BlockSpec((2 * nH, N, BT), lambda b, t: (0, 0, b)),
            pl.BlockSpec((nH * M, BT), lambda b, t: (0, b)),
        )
        out_shapes = (
            jax.ShapeDtypeStruct((Tp, OP, Bl), f32),
            jax.ShapeDtypeStruct((C, Bl), f32),
            jax.ShapeDtypeStruct((C, Bl), f32),
            jax.ShapeDtypeStruct((M, N, Bl), f32),
            jax.ShapeDtypeStruct((2 * nH, N, Bl), f32),
            jax.ShapeDtypeStruct((nH * M, Bl), f32),
        )
        fn = pl.pallas_call(
            functools.partial(
                _ntm_kernel,
                num_heads=nH, ctrl=C, mem_m=M, mem_n=N, in_size=IN,
                total_t=T, tt_chunk=TT),
            grid=grid,
            in_specs=in_specs,
            out_specs=out_specs,
            out_shape=out_shapes,
            compiler_params=pltpu.CompilerParams(
                dimension_semantics=("parallel", "arbitrary"),
                vmem_limit_bytes=60 * 1024 * 1024),
        )
        return fn(x_l, reads_l, h_l, c_l, mem_l, w_l,
                  lw_l, lb_l, hww_l, hbb_l, oww_l, obb_l)

    if ndev > 1:
        mesh = Mesh(np.array(jax.devices()[:ndev]), ("d",))
        sh_b2 = P(None, "d")
        sh_b3 = P(None, None, "d")
        rep = P()
        run = _shard_map(
            run_local, mesh=mesh,
            in_specs=(sh_b3, sh_b2, sh_b2, sh_b2, sh_b3, sh_b3,
                      rep, rep, rep, rep, rep, rep),
            out_specs=(sh_b3, sh_b2, sh_b2, sh_b3, sh_b3, sh_b2),
            check_rep=False)
    else:
        run = run_local

    y_seq, h_new, c_new, mem_new, w_new, reads_new = run(
        x_p, reads_p, h_p, c_p, mem_p, w_p,
        lstm_wT, lstm_bT, head_wT, head_bT, out_wT, out_bT)

    new_state = {
        "h": h_new[:, :B].T,
        "c": c_new[:, :B].T,
        "memory": jnp.transpose(mem_new[:, :, :B], (2, 1, 0)),
        "prev_w": jnp.transpose(w_new[:, :, :B], (0, 2, 1)),
        "prev_reads": jnp.transpose(
            reads_new[:, :B].reshape(nH, M, B), (0, 2, 1)),
    }
    return (jnp.transpose(y_seq[:T, :output_size, :B], (0, 2, 1)), new_state)
